# Initial kernel scaffold; baseline (speedup 1.0000x reference)
import jax
import jax.numpy as jnp
from jax.experimental import pallas as pl


def kernel(x, edge_index, W1, a_src1, a_dst1, b1, g1, be1, W2, a_src2, a_dst2, b2, g2, be2, mW1, mb1, mW2, mb2):
    raise NotImplementedError("write your pallas kernel here")


# speedup vs baseline: 16.8697x; 16.8697x over previous
import functools
import math

import jax
import jax.numpy as jnp
from jax import lax
from jax.experimental import pallas as pl
from jax.experimental.pallas import tpu as pltpu
from jax.experimental.pallas import tpu_sc as plsc

NN = 10000
NP = 10240
EE = 320000
EP = 331776
BB = 128
DIN = 128
HID = 64
HDS = 8
RSIG = float(1.0 / math.sqrt(1.0 + 1e-5))


def _elu(t):
    return jnp.where(t > 0, t, jnp.exp(jnp.minimum(t, 0.0)) - 1.0)


def _tc1_body(x_ref, w1_ref, as_ref, ad_ref, h_out, acat_out):
    hb = jnp.dot(x_ref[...], w1_ref[...], preferred_element_type=jnp.float32)
    iota = lax.broadcasted_iota(jnp.int32, (1, 16), 1)
    acat = jnp.zeros((x_ref.shape[0], 16), jnp.float32)
    for h in range(HDS):
        hh = hb[:, h * HID:(h + 1) * HID]
        h_out[h] = hh
        asv = jnp.sum(hh * as_ref[h][None, :], axis=1)
        adv = jnp.sum(hh * ad_ref[h][None, :], axis=1)
        acat = acat + jnp.where(iota == h, asv[:, None], 0.0)
        acat = acat + jnp.where(iota == HDS + h, adv[:, None], 0.0)
    acat_out[...] = acat


def _tc1(x, W1, a_src1, a_dst1):
    nb = NP // 512
    return pl.pallas_call(
        _tc1_body,
        grid=(nb,),
        in_specs=[
            pl.BlockSpec((512, DIN), lambda i: (i, 0)),
            pl.BlockSpec((DIN, HDS * HID), lambda i: (0, 0)),
            pl.BlockSpec((HDS, HID), lambda i: (0, 0)),
            pl.BlockSpec((HDS, HID), lambda i: (0, 0)),
        ],
        out_specs=[
            pl.BlockSpec((HDS, 512, HID), lambda i: (0, i, 0)),
            pl.BlockSpec((512, 16), lambda i: (i, 0)),
        ],
        out_shape=[
            jax.ShapeDtypeStruct((HDS, NN, HID), jnp.float32),
            jax.ShapeDtypeStruct((NN, 16), jnp.float32),
        ],
    )(x, W1, a_src1, a_dst1)


def _tc2_body(num_ref, den_ref, b1_ref, g1_ref, be1_ref, w2_ref,
              as2_ref, ad2_ref, h2_out, acat_out):
    den = den_ref[0] + den_ref[1]
    acc = jnp.zeros((num_ref.shape[1], HID), jnp.float32)
    for h in range(HDS):
        t = num_ref[h] / (den[:, h][:, None] + 1e-16)
        t = t + b1_ref[h * HID:(h + 1) * HID][None, :]
        t = t * (g1_ref[h * HID:(h + 1) * HID][None, :] * RSIG)
        t = t + be1_ref[h * HID:(h + 1) * HID][None, :]
        t = _elu(t)
        acc = acc + jnp.dot(t, w2_ref[h * HID:(h + 1) * HID, :],
                            preferred_element_type=jnp.float32)
    h2_out[...] = acc
    asv = jnp.sum(acc * as2_ref[...], axis=1)
    adv = jnp.sum(acc * ad2_ref[...], axis=1)
    iota = lax.broadcasted_iota(jnp.int32, (1, 16), 1)
    acat_out[...] = (jnp.where(iota == 0, asv[:, None], 0.0)
                     + jnp.where(iota == 1, adv[:, None], 0.0))


def _tc2(num1, den1, b1, g1, be1, W2, a_src2, a_dst2):
    nb = NP // 512
    return pl.pallas_call(
        _tc2_body,
        grid=(nb,),
        in_specs=[
            pl.BlockSpec((HDS, 512, HID), lambda i: (0, i, 0)),
            pl.BlockSpec((2, 512, 16), lambda i: (0, i, 0)),
            pl.BlockSpec((HDS * HID,), lambda i: (0,)),
            pl.BlockSpec((HDS * HID,), lambda i: (0,)),
            pl.BlockSpec((HDS * HID,), lambda i: (0,)),
            pl.BlockSpec((HDS * HID, HID), lambda i: (0, 0)),
            pl.BlockSpec((1, HID), lambda i: (0, 0)),
            pl.BlockSpec((1, HID), lambda i: (0, 0)),
        ],
        out_specs=[
            pl.BlockSpec((512, HID), lambda i: (i, 0)),
            pl.BlockSpec((512, 16), lambda i: (i, 0)),
        ],
        out_shape=[
            jax.ShapeDtypeStruct((NN, HID), jnp.float32),
            jax.ShapeDtypeStruct((NN, 16), jnp.float32),
        ],
    )(num1, den1, b1, g1, be1, W2, a_src2, a_dst2)


def _tc3_body(num_ref, den_ref, b2_ref, g2_ref, be2_ref, x_ref,
              mw1_ref, mb1_ref, mw2_ref, mb2_ref, y_out):
    num = num_ref[0] + num_ref[1]
    den = den_ref[0, :, 0] + den_ref[1, :, 0]
    t = num / (den[:, None] + 1e-16) + b2_ref[...][None, :]
    t = t * (g2_ref[...][None, :] * RSIG) + be2_ref[...][None, :]
    t = _elu(t)
    m = (jnp.dot(t, mw1_ref[0:HID, :], preferred_element_type=jnp.float32)
         + jnp.dot(x_ref[...], mw1_ref[HID:HID + DIN, :],
                   preferred_element_type=jnp.float32)
         + mb1_ref[...][None, :])
    m = jnp.maximum(m, 0.0)
    y = jnp.sum(m * mw2_ref[...][None, :], axis=1) + mb2_ref[...]
    y_out[...] = y


def _tc3(num2, den2, b2, g2, be2, x, mW1, mb1, mw2v, mb2):
    nb = NP // 512
    return pl.pallas_call(
        _tc3_body,
        grid=(nb,),
        in_specs=[
            pl.BlockSpec((2, 512, HID), lambda i: (0, i, 0)),
            pl.BlockSpec((2, 512, 16), lambda i: (0, i, 0)),
            pl.BlockSpec((HID,), lambda i: (0,)),
            pl.BlockSpec((HID,), lambda i: (0,)),
            pl.BlockSpec((HID,), lambda i: (0,)),
            pl.BlockSpec((512, DIN), lambda i: (i, 0)),
            pl.BlockSpec((HID + DIN, HID), lambda i: (0, 0)),
            pl.BlockSpec((HID,), lambda i: (0,)),
            pl.BlockSpec((HID,), lambda i: (0,)),
            pl.BlockSpec((1,), lambda i: (0,)),
        ],
        out_specs=pl.BlockSpec((512,), lambda i: (i,)),
        out_shape=jax.ShapeDtypeStruct((NP,), jnp.float32),
    )(num2, den2, b2, g2, be2, x, mW1, mb1, mw2v, mb2)


def _bcast(v, lane):
    idx = jnp.full((16,), lane, jnp.int32)
    return v.at[idx].get(mode="promise_in_bounds")


def _sc_weights(acatp, src, dst, heads):
    mesh = plsc.VectorSubcoreMesh(core_axis_name="c", subcore_axis_name="s")
    chunk = EP // 32
    nblk = chunk // BB
    slab = NP // 16

    def body(acat_hbm, src_hbm, dst_hbm, wt_hbm, den_hbm,
             sidx, didx, asb, adb, wb, zrw, sem1, sem2, den_sh):
        c = lax.axis_index("c")
        s = lax.axis_index("s")
        wid = s * 2 + c
        for r in range(BB):
            zrw[r] = jnp.zeros((16,), jnp.float32)
        for j in range(slab // BB):
            pltpu.sync_copy(zrw, den_sh.at[pl.ds(s * slab + j * BB, BB)])
        plsc.subcore_barrier()
        shift = (jnp.arange(16, dtype=jnp.int32) % heads) + heads

        def blk(b, carry):
            base = wid * chunk + b * BB
            pltpu.sync_copy(src_hbm.at[pl.ds(base, BB)], sidx)
            pltpu.sync_copy(dst_hbm.at[pl.ds(base, BB)], didx)
            cp1 = pltpu.async_copy(acat_hbm.at[sidx], asb, sem1)
            cp2 = pltpu.async_copy(acat_hbm.at[didx], adb, sem2)
            cp1.wait()
            cp2.wait()
            for e in range(BB):
                ev = asb[e] + adb[e].at[shift].get(mode="promise_in_bounds")
                ev = jnp.where(ev >= 0, ev, 0.2 * ev)
                wb[e] = jnp.exp(ev)
            pltpu.sync_copy(wb, wt_hbm.at[pl.ds(base, BB)])
            pltpu.sync_copy(wb, den_sh.at[didx], add=True)
            return carry

        lax.fori_loop(0, nblk, blk, 0)
        plsc.subcore_barrier()
        pltpu.sync_copy(den_sh.at[pl.ds(s * slab, slab)],
                        den_hbm.at[c, pl.ds(s * slab, slab)])

    kern = pl.kernel(
        body,
        out_type=[
            jax.ShapeDtypeStruct((EP, 16), jnp.float32),
            jax.ShapeDtypeStruct((2, NP, 16), jnp.float32),
        ],
        mesh=mesh,
        compiler_params=pltpu.CompilerParams(use_tc_tiling_on_sc=False),
        scratch_types=[
            pltpu.VMEM((BB,), jnp.int32),
            pltpu.VMEM((BB,), jnp.int32),
            pltpu.VMEM((BB, 16), jnp.float32),
            pltpu.VMEM((BB, 16), jnp.float32),
            pltpu.VMEM((BB, 16), jnp.float32),
            pltpu.VMEM((BB, 16), jnp.float32),
            pltpu.SemaphoreType.DMA,
            pltpu.SemaphoreType.DMA,
            pltpu.VMEM_SHARED((NP, 16), jnp.float32),
        ],
    )
    return kern(acatp, src, dst)


def _sc_agg_heads(hflat, wt, src, dst):
    mesh = plsc.VectorSubcoreMesh(core_axis_name="c", subcore_axis_name="s")
    chunk = EP // 16
    nblk = chunk // BB
    slab = NP // 16

    def body(hflat_hbm, wt_hbm, src_hbm, dst_hbm, num_hbm,
             sidx, didx, gidx, rows, wv, zrow, sem, acc_sh):
        c = lax.axis_index("c")
        s = lax.axis_index("s")
        for r in range(BB):
            for k in range(HID // 16):
                zrow[r, pl.ds(k * 16, 16)] = jnp.zeros((16,), jnp.float32)
        for j in range(slab // BB):
            pltpu.sync_copy(zrow, acc_sh.at[pl.ds(s * slab + j * BB, BB)])
        plsc.subcore_barrier()
        for hp in range(4):
            h = c * 4 + hp
            hoff = h * NN

            def blk(b, carry):
                base = s * chunk + b * BB
                pltpu.sync_copy(src_hbm.at[pl.ds(base, BB)], sidx)
                pltpu.sync_copy(dst_hbm.at[pl.ds(base, BB)], didx)
                for i in range(BB // 16):
                    gidx[pl.ds(i * 16, 16)] = sidx[pl.ds(i * 16, 16)] + hoff
                pltpu.async_copy(hflat_hbm.at[gidx], rows, sem).wait()
                pltpu.sync_copy(wt_hbm.at[pl.ds(base, BB)], wv)
                for r in range(BB):
                    wsp = _bcast(wv[r], h)
                    for k in range(HID // 16):
                        rows[r, pl.ds(k * 16, 16)] = (
                            rows[r, pl.ds(k * 16, 16)] * wsp)
                pltpu.sync_copy(rows, acc_sh.at[didx], add=True)
                return carry

            lax.fori_loop(0, nblk, blk, 0)
            plsc.subcore_barrier()
            pltpu.sync_copy(acc_sh.at[pl.ds(s * slab, slab)],
                            num_hbm.at[h, pl.ds(s * slab, slab)])
            if hp < 3:
                for j in range(slab // BB):
                    pltpu.sync_copy(zrow, acc_sh.at[pl.ds(s * slab + j * BB, BB)])
                plsc.subcore_barrier()

    kern = pl.kernel(
        body,
        out_type=jax.ShapeDtypeStruct((HDS, NP, HID), jnp.float32),
        mesh=mesh,
        compiler_params=pltpu.CompilerParams(use_tc_tiling_on_sc=False),
        scratch_types=[
            pltpu.VMEM((BB,), jnp.int32),
            pltpu.VMEM((BB,), jnp.int32),
            pltpu.VMEM((BB,), jnp.int32),
            pltpu.VMEM((BB, HID), jnp.float32),
            pltpu.VMEM((BB, 16), jnp.float32),
            pltpu.VMEM((BB, HID), jnp.float32),
            pltpu.SemaphoreType.DMA,
            pltpu.VMEM_SHARED((NP, HID), jnp.float32),
        ],
    )
    return kern(hflat, wt, src, dst)


def _sc_agg_single(h2, wt, src, dst):
    mesh = plsc.VectorSubcoreMesh(core_axis_name="c", subcore_axis_name="s")
    chunk = EP // 32
    nblk = chunk // BB
    slab = NP // 16

    def body(h2_hbm, wt_hbm, src_hbm, dst_hbm, num_hbm,
             sidx, didx, rows, wv, zrow, sem, acc_sh):
        c = lax.axis_index("c")
        s = lax.axis_index("s")
        wid = s * 2 + c
        for r in range(BB):
            for k in range(HID // 16):
                zrow[r, pl.ds(k * 16, 16)] = jnp.zeros((16,), jnp.float32)
        for j in range(slab // BB):
            pltpu.sync_copy(zrow, acc_sh.at[pl.ds(s * slab + j * BB, BB)])
        plsc.subcore_barrier()

        def blk(b, carry):
            base = wid * chunk + b * BB
            pltpu.sync_copy(src_hbm.at[pl.ds(base, BB)], sidx)
            pltpu.sync_copy(dst_hbm.at[pl.ds(base, BB)], didx)
            pltpu.async_copy(h2_hbm.at[sidx], rows, sem).wait()
            pltpu.sync_copy(wt_hbm.at[pl.ds(base, BB)], wv)
            for r in range(BB):
                wsp = _bcast(wv[r], 0)
                for k in range(HID // 16):
                    rows[r, pl.ds(k * 16, 16)] = (
                        rows[r, pl.ds(k * 16, 16)] * wsp)
            pltpu.sync_copy(rows, acc_sh.at[didx], add=True)
            return carry

        lax.fori_loop(0, nblk, blk, 0)
        plsc.subcore_barrier()
        pltpu.sync_copy(acc_sh.at[pl.ds(s * slab, slab)],
                        num_hbm.at[c, pl.ds(s * slab, slab)])

    kern = pl.kernel(
        body,
        out_type=jax.ShapeDtypeStruct((2, NP, HID), jnp.float32),
        mesh=mesh,
        compiler_params=pltpu.CompilerParams(use_tc_tiling_on_sc=False),
        scratch_types=[
            pltpu.VMEM((BB,), jnp.int32),
            pltpu.VMEM((BB,), jnp.int32),
            pltpu.VMEM((BB, HID), jnp.float32),
            pltpu.VMEM((BB, 16), jnp.float32),
            pltpu.VMEM((BB, HID), jnp.float32),
            pltpu.SemaphoreType.DMA,
            pltpu.VMEM_SHARED((NP, HID), jnp.float32),
        ],
    )
    return kern(h2, wt, src, dst)


def kernel(x, edge_index, W1, a_src1, a_dst1, b1, g1, be1,
           W2, a_src2, a_dst2, b2, g2, be2, mW1, mb1, mW2, mb2):
    src0 = edge_index[0].astype(jnp.int32)
    dst0 = edge_index[1].astype(jnp.int32)
    loop = jnp.arange(NN, dtype=jnp.int32)
    npad = EP - (EE + NN)
    src = jnp.concatenate([src0, loop, jnp.zeros((npad,), jnp.int32)])
    dst = jnp.concatenate([dst0, loop, jnp.full((npad,), NN, jnp.int32)])

    h1t, acat1 = _tc1(x, W1, a_src1, a_dst1)
    acat1p = jnp.pad(acat1, ((0, NP - NN), (0, 0)))
    w1t, den1 = _sc_weights(acat1p, src, dst, HDS)
    num1 = _sc_agg_heads(h1t.reshape(HDS * NN, HID), w1t, src, dst)

    h2, acat2 = _tc2(num1, den1, b1, g1, be1, W2, a_src2, a_dst2)
    acat2p = jnp.pad(acat2, ((0, NP - NN), (0, 0)))
    w2t, den2 = _sc_weights(acat2p, src, dst, 1)
    num2 = _sc_agg_single(h2, w2t, src, dst)

    y = _tc3(num2, den2, b2, g2, be2, x, mW1, mb1,
             mW2.reshape(HID), mb2)
    return y[:NN]


# speedup vs baseline: 28.0927x; 1.6653x over previous
import functools
import math

import jax
import jax.numpy as jnp
from jax import lax
from jax.experimental import pallas as pl
from jax.experimental.pallas import tpu as pltpu
from jax.experimental.pallas import tpu_sc as plsc

NN = 10000
NP = 10240
EE = 320000
EP = 331776
BB = 128
DIN = 128
HID = 64
HDS = 8
RSIG = float(1.0 / math.sqrt(1.0 + 1e-5))


def _elu(t):
    return jnp.where(t > 0, t, jnp.exp(jnp.minimum(t, 0.0)) - 1.0)


def _tc1_body(x_ref, w1_ref, as_ref, ad_ref, h_out, acat_out):
    hb = jnp.dot(x_ref[...], w1_ref[...], preferred_element_type=jnp.float32)
    iota = lax.broadcasted_iota(jnp.int32, (1, 16), 1)
    acat = jnp.zeros((x_ref.shape[0], 16), jnp.float32)
    for h in range(HDS):
        hh = hb[:, h * HID:(h + 1) * HID]
        h_out[h] = hh
        asv = jnp.sum(hh * as_ref[h][None, :], axis=1)
        adv = jnp.sum(hh * ad_ref[h][None, :], axis=1)
        acat = acat + jnp.where(iota == h, asv[:, None], 0.0)
        acat = acat + jnp.where(iota == HDS + h, adv[:, None], 0.0)
    acat_out[...] = acat


def _tc1(x, W1, a_src1, a_dst1):
    nb = NP // 512
    return pl.pallas_call(
        _tc1_body,
        grid=(nb,),
        in_specs=[
            pl.BlockSpec((512, DIN), lambda i: (i, 0)),
            pl.BlockSpec((DIN, HDS * HID), lambda i: (0, 0)),
            pl.BlockSpec((HDS, HID), lambda i: (0, 0)),
            pl.BlockSpec((HDS, HID), lambda i: (0, 0)),
        ],
        out_specs=[
            pl.BlockSpec((HDS, 512, HID), lambda i: (0, i, 0)),
            pl.BlockSpec((512, 16), lambda i: (i, 0)),
        ],
        out_shape=[
            jax.ShapeDtypeStruct((HDS, NN, HID), jnp.float32),
            jax.ShapeDtypeStruct((NN, 16), jnp.float32),
        ],
    )(x, W1, a_src1, a_dst1)


def _tc2_body(num_ref, den_ref, b1_ref, g1_ref, be1_ref, w2_ref,
              as2_ref, ad2_ref, h2_out, acat_out):
    den = den_ref[0] + den_ref[1]
    acc = jnp.zeros((num_ref.shape[1], HID), jnp.float32)
    for h in range(HDS):
        t = num_ref[h] / (den[:, h][:, None] + 1e-16)
        t = t + b1_ref[h * HID:(h + 1) * HID][None, :]
        t = t * (g1_ref[h * HID:(h + 1) * HID][None, :] * RSIG)
        t = t + be1_ref[h * HID:(h + 1) * HID][None, :]
        t = _elu(t)
        acc = acc + jnp.dot(t, w2_ref[h * HID:(h + 1) * HID, :],
                            preferred_element_type=jnp.float32)
    h2_out[...] = acc
    asv = jnp.sum(acc * as2_ref[...], axis=1)
    adv = jnp.sum(acc * ad2_ref[...], axis=1)
    iota = lax.broadcasted_iota(jnp.int32, (1, 16), 1)
    acat_out[...] = (jnp.where(iota == 0, asv[:, None], 0.0)
                     + jnp.where(iota == 1, adv[:, None], 0.0))


def _tc2(num1, den1, b1, g1, be1, W2, a_src2, a_dst2):
    nb = NP // 512
    return pl.pallas_call(
        _tc2_body,
        grid=(nb,),
        in_specs=[
            pl.BlockSpec((HDS, 512, HID), lambda i: (0, i, 0)),
            pl.BlockSpec((2, 512, 16), lambda i: (0, i, 0)),
            pl.BlockSpec((HDS * HID,), lambda i: (0,)),
            pl.BlockSpec((HDS * HID,), lambda i: (0,)),
            pl.BlockSpec((HDS * HID,), lambda i: (0,)),
            pl.BlockSpec((HDS * HID, HID), lambda i: (0, 0)),
            pl.BlockSpec((1, HID), lambda i: (0, 0)),
            pl.BlockSpec((1, HID), lambda i: (0, 0)),
        ],
        out_specs=[
            pl.BlockSpec((512, HID), lambda i: (i, 0)),
            pl.BlockSpec((512, 16), lambda i: (i, 0)),
        ],
        out_shape=[
            jax.ShapeDtypeStruct((NN, HID), jnp.float32),
            jax.ShapeDtypeStruct((NN, 16), jnp.float32),
        ],
    )(num1, den1, b1, g1, be1, W2, a_src2, a_dst2)


def _tc3_body(num_ref, den_ref, b2_ref, g2_ref, be2_ref, x_ref,
              mw1_ref, mb1_ref, mw2_ref, mb2_ref, y_out):
    num = num_ref[0] + num_ref[1]
    den = den_ref[0, :, 0] + den_ref[1, :, 0]
    t = num / (den[:, None] + 1e-16) + b2_ref[...][None, :]
    t = t * (g2_ref[...][None, :] * RSIG) + be2_ref[...][None, :]
    t = _elu(t)
    m = (jnp.dot(t, mw1_ref[0:HID, :], preferred_element_type=jnp.float32)
         + jnp.dot(x_ref[...], mw1_ref[HID:HID + DIN, :],
                   preferred_element_type=jnp.float32)
         + mb1_ref[...][None, :])
    m = jnp.maximum(m, 0.0)
    y = jnp.sum(m * mw2_ref[...][None, :], axis=1) + mb2_ref[...]
    y_out[...] = y


def _tc3(num2, den2, b2, g2, be2, x, mW1, mb1, mw2v, mb2):
    nb = NP // 512
    return pl.pallas_call(
        _tc3_body,
        grid=(nb,),
        in_specs=[
            pl.BlockSpec((2, 512, HID), lambda i: (0, i, 0)),
            pl.BlockSpec((2, 512, 16), lambda i: (0, i, 0)),
            pl.BlockSpec((HID,), lambda i: (0,)),
            pl.BlockSpec((HID,), lambda i: (0,)),
            pl.BlockSpec((HID,), lambda i: (0,)),
            pl.BlockSpec((512, DIN), lambda i: (i, 0)),
            pl.BlockSpec((HID + DIN, HID), lambda i: (0, 0)),
            pl.BlockSpec((HID,), lambda i: (0,)),
            pl.BlockSpec((HID,), lambda i: (0,)),
            pl.BlockSpec((1,), lambda i: (0,)),
        ],
        out_specs=pl.BlockSpec((512,), lambda i: (i,)),
        out_shape=jax.ShapeDtypeStruct((NP,), jnp.float32),
    )(num2, den2, b2, g2, be2, x, mW1, mb1, mw2v, mb2)


def _bcast(v, lane):
    idx = jnp.full((16,), lane, jnp.int32)
    return v.at[idx].get(mode="promise_in_bounds")


def _sc_weights(acatp, src, dst, heads):
    mesh = plsc.VectorSubcoreMesh(core_axis_name="c", subcore_axis_name="s")
    chunk = EP // 32
    nblk = chunk // BB
    slab = NP // 16

    def body(acat_hbm, src_hbm, dst_hbm, wt_hbm, den_hbm,
             sidx, didx, asb, adb, wb, zrw, sem1, sem2, semi1, semi2, den_sh):
        c = lax.axis_index("c")
        s = lax.axis_index("s")
        wid = s * 2 + c
        for r in range(BB):
            zrw[r] = jnp.zeros((16,), jnp.float32)
        for j in range(slab // BB):
            pltpu.sync_copy(zrw, den_sh.at[pl.ds(s * slab + j * BB, BB)])
        plsc.subcore_barrier()
        shift = (jnp.arange(16, dtype=jnp.int32) % heads) + heads

        def blk(b, carry):
            base = wid * chunk + b * BB
            cps = pltpu.async_copy(src_hbm.at[pl.ds(base, BB)], sidx, semi1)
            cpd = pltpu.async_copy(dst_hbm.at[pl.ds(base, BB)], didx, semi2)
            cps.wait()
            cp1 = pltpu.async_copy(acat_hbm.at[sidx], asb, sem1)
            cpd.wait()
            cp2 = pltpu.async_copy(acat_hbm.at[didx], adb, sem2)
            cp1.wait()
            cp2.wait()
            for e in range(BB):
                ev = asb[e] + adb[e].at[shift].get(mode="promise_in_bounds")
                ev = jnp.where(ev >= 0, ev, 0.2 * ev)
                wb[e] = jnp.exp(ev)
            pltpu.sync_copy(wb, wt_hbm.at[pl.ds(base, BB)])
            pltpu.sync_copy(wb, den_sh.at[didx], add=True)
            return carry

        lax.fori_loop(0, nblk, blk, 0)
        plsc.subcore_barrier()
        pltpu.sync_copy(den_sh.at[pl.ds(s * slab, slab)],
                        den_hbm.at[c, pl.ds(s * slab, slab)])

    kern = pl.kernel(
        body,
        out_type=[
            jax.ShapeDtypeStruct((EP, 16), jnp.float32),
            jax.ShapeDtypeStruct((2, NP, 16), jnp.float32),
        ],
        mesh=mesh,
        compiler_params=pltpu.CompilerParams(use_tc_tiling_on_sc=False),
        scratch_types=[
            pltpu.VMEM((BB,), jnp.int32),
            pltpu.VMEM((BB,), jnp.int32),
            pltpu.VMEM((BB, 16), jnp.float32),
            pltpu.VMEM((BB, 16), jnp.float32),
            pltpu.VMEM((BB, 16), jnp.float32),
            pltpu.VMEM((BB, 16), jnp.float32),
            pltpu.SemaphoreType.DMA,
            pltpu.SemaphoreType.DMA,
            pltpu.SemaphoreType.DMA,
            pltpu.SemaphoreType.DMA,
            pltpu.VMEM_SHARED((NP, 16), jnp.float32),
        ],
    )
    return kern(acatp, src, dst)


def _sc_agg_heads(hflat, wt, src, dst):
    mesh = plsc.VectorSubcoreMesh(core_axis_name="c", subcore_axis_name="s")
    chunk = EP // 16
    nblk = chunk // BB
    slab = NP // 16

    def body(hflat_hbm, wt_hbm, src_hbm, dst_hbm, num_hbm,
             sidx, didx, gidx0, gidx1, rows0, rows1, wv, zrow,
             semi1, semi2, semw, semg0, semg1, acc0, acc1):
        c = lax.axis_index("c")
        s = lax.axis_index("s")
        for r in range(BB):
            for k in range(HID // 16):
                zrow[r, pl.ds(k * 16, 16)] = jnp.zeros((16,), jnp.float32)

        def zero_accs():
            for j in range(slab // BB):
                pltpu.sync_copy(zrow, acc0.at[pl.ds(s * slab + j * BB, BB)])
                pltpu.sync_copy(zrow, acc1.at[pl.ds(s * slab + j * BB, BB)])

        zero_accs()
        plsc.subcore_barrier()
        for hp in range(2):
            h0 = c * 4 + 2 * hp
            off0 = h0 * NN
            off1 = off0 + NN

            def blk(b, carry):
                base = s * chunk + b * BB
                cps = pltpu.async_copy(src_hbm.at[pl.ds(base, BB)], sidx, semi1)
                cpd = pltpu.async_copy(dst_hbm.at[pl.ds(base, BB)], didx, semi2)
                cpw = pltpu.async_copy(wt_hbm.at[pl.ds(base, BB)], wv, semw)
                cps.wait()
                for i in range(BB // 16):
                    sv = sidx[pl.ds(i * 16, 16)]
                    gidx0[pl.ds(i * 16, 16)] = sv + off0
                    gidx1[pl.ds(i * 16, 16)] = sv + off1
                g0 = pltpu.async_copy(hflat_hbm.at[gidx0], rows0, semg0)
                g1 = pltpu.async_copy(hflat_hbm.at[gidx1], rows1, semg1)
                cpw.wait()
                g0.wait()
                for r in range(BB):
                    wsp = _bcast(wv[r], h0)
                    for k in range(HID // 16):
                        rows0[r, pl.ds(k * 16, 16)] = (
                            rows0[r, pl.ds(k * 16, 16)] * wsp)
                cpd.wait()
                pltpu.sync_copy(rows0, acc0.at[didx], add=True)
                g1.wait()
                for r in range(BB):
                    wsp = _bcast(wv[r], h0 + 1)
                    for k in range(HID // 16):
                        rows1[r, pl.ds(k * 16, 16)] = (
                            rows1[r, pl.ds(k * 16, 16)] * wsp)
                pltpu.sync_copy(rows1, acc1.at[didx], add=True)
                return carry

            lax.fori_loop(0, nblk, blk, 0)
            plsc.subcore_barrier()
            pltpu.sync_copy(acc0.at[pl.ds(s * slab, slab)],
                            num_hbm.at[h0, pl.ds(s * slab, slab)])
            pltpu.sync_copy(acc1.at[pl.ds(s * slab, slab)],
                            num_hbm.at[h0 + 1, pl.ds(s * slab, slab)])
            if hp == 0:
                zero_accs()
                plsc.subcore_barrier()

    kern = pl.kernel(
        body,
        out_type=jax.ShapeDtypeStruct((HDS, NP, HID), jnp.float32),
        mesh=mesh,
        compiler_params=pltpu.CompilerParams(use_tc_tiling_on_sc=False),
        scratch_types=[
            pltpu.VMEM((BB,), jnp.int32),
            pltpu.VMEM((BB,), jnp.int32),
            pltpu.VMEM((BB,), jnp.int32),
            pltpu.VMEM((BB,), jnp.int32),
            pltpu.VMEM((BB, HID), jnp.float32),
            pltpu.VMEM((BB, HID), jnp.float32),
            pltpu.VMEM((BB, 16), jnp.float32),
            pltpu.VMEM((BB, HID), jnp.float32),
            pltpu.SemaphoreType.DMA,
            pltpu.SemaphoreType.DMA,
            pltpu.SemaphoreType.DMA,
            pltpu.SemaphoreType.DMA,
            pltpu.SemaphoreType.DMA,
            pltpu.VMEM_SHARED((NP, HID), jnp.float32),
            pltpu.VMEM_SHARED((NP, HID), jnp.float32),
        ],
    )
    return kern(hflat, wt, src, dst)


def _sc_agg_single(h2, wt, src, dst):
    mesh = plsc.VectorSubcoreMesh(core_axis_name="c", subcore_axis_name="s")
    chunk = EP // 32
    nblk = chunk // BB
    slab = NP // 16

    def body(h2_hbm, wt_hbm, src_hbm, dst_hbm, num_hbm,
             sidx, didx, rows, wv, zrow, sem, semi1, semi2, semw, acc_sh):
        c = lax.axis_index("c")
        s = lax.axis_index("s")
        wid = s * 2 + c
        for r in range(BB):
            for k in range(HID // 16):
                zrow[r, pl.ds(k * 16, 16)] = jnp.zeros((16,), jnp.float32)
        for j in range(slab // BB):
            pltpu.sync_copy(zrow, acc_sh.at[pl.ds(s * slab + j * BB, BB)])
        plsc.subcore_barrier()

        def blk(b, carry):
            base = wid * chunk + b * BB
            cps = pltpu.async_copy(src_hbm.at[pl.ds(base, BB)], sidx, semi1)
            cpd = pltpu.async_copy(dst_hbm.at[pl.ds(base, BB)], didx, semi2)
            cpw = pltpu.async_copy(wt_hbm.at[pl.ds(base, BB)], wv, semw)
            cps.wait()
            g0 = pltpu.async_copy(h2_hbm.at[sidx], rows, sem)
            cpw.wait()
            g0.wait()
            for r in range(BB):
                wsp = _bcast(wv[r], 0)
                for k in range(HID // 16):
                    rows[r, pl.ds(k * 16, 16)] = (
                        rows[r, pl.ds(k * 16, 16)] * wsp)
            cpd.wait()
            pltpu.sync_copy(rows, acc_sh.at[didx], add=True)
            return carry

        lax.fori_loop(0, nblk, blk, 0)
        plsc.subcore_barrier()
        pltpu.sync_copy(acc_sh.at[pl.ds(s * slab, slab)],
                        num_hbm.at[c, pl.ds(s * slab, slab)])

    kern = pl.kernel(
        body,
        out_type=jax.ShapeDtypeStruct((2, NP, HID), jnp.float32),
        mesh=mesh,
        compiler_params=pltpu.CompilerParams(use_tc_tiling_on_sc=False),
        scratch_types=[
            pltpu.VMEM((BB,), jnp.int32),
            pltpu.VMEM((BB,), jnp.int32),
            pltpu.VMEM((BB, HID), jnp.float32),
            pltpu.VMEM((BB, 16), jnp.float32),
            pltpu.VMEM((BB, HID), jnp.float32),
            pltpu.SemaphoreType.DMA,
            pltpu.SemaphoreType.DMA,
            pltpu.SemaphoreType.DMA,
            pltpu.SemaphoreType.DMA,
            pltpu.VMEM_SHARED((NP, HID), jnp.float32),
        ],
    )
    return kern(h2, wt, src, dst)


def kernel(x, edge_index, W1, a_src1, a_dst1, b1, g1, be1,
           W2, a_src2, a_dst2, b2, g2, be2, mW1, mb1, mW2, mb2):
    src0 = edge_index[0].astype(jnp.int32)
    dst0 = edge_index[1].astype(jnp.int32)
    loop = jnp.arange(NN, dtype=jnp.int32)
    npad = EP - (EE + NN)
    src = jnp.concatenate([src0, loop, jnp.zeros((npad,), jnp.int32)])
    dst = jnp.concatenate([dst0, loop, jnp.full((npad,), NN, jnp.int32)])

    h1t, acat1 = _tc1(x, W1, a_src1, a_dst1)
    acat1p = jnp.pad(acat1, ((0, NP - NN), (0, 0)))
    w1t, den1 = _sc_weights(acat1p, src, dst, HDS)
    num1 = _sc_agg_heads(h1t.reshape(HDS * NN, HID), w1t, src, dst)

    h2, acat2 = _tc2(num1, den1, b1, g1, be1, W2, a_src2, a_dst2)
    acat2p = jnp.pad(acat2, ((0, NP - NN), (0, 0)))
    w2t, den2 = _sc_weights(acat2p, src, dst, 1)
    num2 = _sc_agg_single(h2, w2t, src, dst)

    y = _tc3(num2, den2, b2, g2, be2, x, mW1, mb1,
             mW2.reshape(HID), mb2)
    return y[:NN]


# speedup vs baseline: 28.4072x; 1.0112x over previous
import functools
import math

import jax
import jax.numpy as jnp
from jax import lax
from jax.experimental import pallas as pl
from jax.experimental.pallas import tpu as pltpu
from jax.experimental.pallas import tpu_sc as plsc

NN = 10000
NP = 10240
EE = 320000
EP = 331776
BB = 128
DIN = 128
HID = 64
HDS = 8
RSIG = float(1.0 / math.sqrt(1.0 + 1e-5))


def _elu(t):
    return jnp.where(t > 0, t, jnp.exp(jnp.minimum(t, 0.0)) - 1.0)


def _tc1_body(x_ref, w1_ref, as_ref, ad_ref, h_out, acat_out):
    hb = jnp.dot(x_ref[...], w1_ref[...], preferred_element_type=jnp.float32)
    iota = lax.broadcasted_iota(jnp.int32, (1, 16), 1)
    acat = jnp.zeros((x_ref.shape[0], 16), jnp.float32)
    for h in range(HDS):
        hh = hb[:, h * HID:(h + 1) * HID]
        h_out[h] = hh
        asv = jnp.sum(hh * as_ref[h][None, :], axis=1)
        adv = jnp.sum(hh * ad_ref[h][None, :], axis=1)
        acat = acat + jnp.where(iota == h, asv[:, None], 0.0)
        acat = acat + jnp.where(iota == HDS + h, adv[:, None], 0.0)
    acat_out[...] = acat


def _tc1(x, W1, a_src1, a_dst1):
    nb = NP // 512
    return pl.pallas_call(
        _tc1_body,
        grid=(nb,),
        in_specs=[
            pl.BlockSpec((512, DIN), lambda i: (i, 0)),
            pl.BlockSpec((DIN, HDS * HID), lambda i: (0, 0)),
            pl.BlockSpec((HDS, HID), lambda i: (0, 0)),
            pl.BlockSpec((HDS, HID), lambda i: (0, 0)),
        ],
        out_specs=[
            pl.BlockSpec((HDS, 512, HID), lambda i: (0, i, 0)),
            pl.BlockSpec((512, 16), lambda i: (i, 0)),
        ],
        out_shape=[
            jax.ShapeDtypeStruct((HDS, NN, HID), jnp.float32),
            jax.ShapeDtypeStruct((NN, 16), jnp.float32),
        ],
    )(x, W1, a_src1, a_dst1)


def _tc2_body(num_ref, den_ref, b1_ref, g1_ref, be1_ref, w2_ref,
              as2_ref, ad2_ref, h2_out, acat_out):
    den = den_ref[0] + den_ref[1]
    acc = jnp.zeros((num_ref.shape[1], HID), jnp.float32)
    for h in range(HDS):
        t = num_ref[h] / (den[:, h][:, None] + 1e-16)
        t = t + b1_ref[h * HID:(h + 1) * HID][None, :]
        t = t * (g1_ref[h * HID:(h + 1) * HID][None, :] * RSIG)
        t = t + be1_ref[h * HID:(h + 1) * HID][None, :]
        t = _elu(t)
        acc = acc + jnp.dot(t, w2_ref[h * HID:(h + 1) * HID, :],
                            preferred_element_type=jnp.float32)
    h2_out[...] = acc
    asv = jnp.sum(acc * as2_ref[...], axis=1)
    adv = jnp.sum(acc * ad2_ref[...], axis=1)
    iota = lax.broadcasted_iota(jnp.int32, (1, 16), 1)
    acat_out[...] = (jnp.where(iota == 0, asv[:, None], 0.0)
                     + jnp.where(iota == 1, adv[:, None], 0.0))


def _tc2(num1, den1, b1, g1, be1, W2, a_src2, a_dst2):
    nb = NP // 512
    return pl.pallas_call(
        _tc2_body,
        grid=(nb,),
        in_specs=[
            pl.BlockSpec((HDS, 512, HID), lambda i: (0, i, 0)),
            pl.BlockSpec((2, 512, 16), lambda i: (0, i, 0)),
            pl.BlockSpec((HDS * HID,), lambda i: (0,)),
            pl.BlockSpec((HDS * HID,), lambda i: (0,)),
            pl.BlockSpec((HDS * HID,), lambda i: (0,)),
            pl.BlockSpec((HDS * HID, HID), lambda i: (0, 0)),
            pl.BlockSpec((1, HID), lambda i: (0, 0)),
            pl.BlockSpec((1, HID), lambda i: (0, 0)),
        ],
        out_specs=[
            pl.BlockSpec((512, HID), lambda i: (i, 0)),
            pl.BlockSpec((512, 16), lambda i: (i, 0)),
        ],
        out_shape=[
            jax.ShapeDtypeStruct((NN, HID), jnp.float32),
            jax.ShapeDtypeStruct((NN, 16), jnp.float32),
        ],
    )(num1, den1, b1, g1, be1, W2, a_src2, a_dst2)


def _tc3_body(num_ref, den_ref, b2_ref, g2_ref, be2_ref, x_ref,
              mw1_ref, mb1_ref, mw2_ref, mb2_ref, y_out):
    num = num_ref[0] + num_ref[1]
    den = den_ref[0, :, 0] + den_ref[1, :, 0]
    t = num / (den[:, None] + 1e-16) + b2_ref[...][None, :]
    t = t * (g2_ref[...][None, :] * RSIG) + be2_ref[...][None, :]
    t = _elu(t)
    m = (jnp.dot(t, mw1_ref[0:HID, :], preferred_element_type=jnp.float32)
         + jnp.dot(x_ref[...], mw1_ref[HID:HID + DIN, :],
                   preferred_element_type=jnp.float32)
         + mb1_ref[...][None, :])
    m = jnp.maximum(m, 0.0)
    y = jnp.sum(m * mw2_ref[...][None, :], axis=1) + mb2_ref[...]
    y_out[...] = y


def _tc3(num2, den2, b2, g2, be2, x, mW1, mb1, mw2v, mb2):
    nb = NP // 512
    return pl.pallas_call(
        _tc3_body,
        grid=(nb,),
        in_specs=[
            pl.BlockSpec((2, 512, HID), lambda i: (0, i, 0)),
            pl.BlockSpec((2, 512, 16), lambda i: (0, i, 0)),
            pl.BlockSpec((HID,), lambda i: (0,)),
            pl.BlockSpec((HID,), lambda i: (0,)),
            pl.BlockSpec((HID,), lambda i: (0,)),
            pl.BlockSpec((512, DIN), lambda i: (i, 0)),
            pl.BlockSpec((HID + DIN, HID), lambda i: (0, 0)),
            pl.BlockSpec((HID,), lambda i: (0,)),
            pl.BlockSpec((HID,), lambda i: (0,)),
            pl.BlockSpec((1,), lambda i: (0,)),
        ],
        out_specs=pl.BlockSpec((512,), lambda i: (i,)),
        out_shape=jax.ShapeDtypeStruct((NP,), jnp.float32),
    )(num2, den2, b2, g2, be2, x, mW1, mb1, mw2v, mb2)


def _bcast(v, lane):
    idx = jnp.full((16,), lane, jnp.int32)
    return v.at[idx].get(mode="promise_in_bounds")


def _sc_weights(acatp, src, dst, heads):
    mesh = plsc.VectorSubcoreMesh(core_axis_name="c", subcore_axis_name="s")
    chunk = EP // 32
    nblk = chunk // BB
    slab = NP // 16

    def body(acat_hbm, src_hbm, dst_hbm, wt_hbm, den_hbm,
             sidx, didx, asb, adb, wb, zrw, sem1, sem2, semi1, semi2, den_sh):
        c = lax.axis_index("c")
        s = lax.axis_index("s")
        wid = s * 2 + c
        for r in range(BB):
            zrw[r] = jnp.zeros((16,), jnp.float32)
        for j in range(slab // BB):
            pltpu.sync_copy(zrw, den_sh.at[pl.ds(s * slab + j * BB, BB)])
        plsc.subcore_barrier()
        shift = (jnp.arange(16, dtype=jnp.int32) % heads) + heads

        def blk(b, carry):
            base = wid * chunk + b * BB
            cps = pltpu.async_copy(src_hbm.at[pl.ds(base, BB)], sidx, semi1)
            cpd = pltpu.async_copy(dst_hbm.at[pl.ds(base, BB)], didx, semi2)
            cps.wait()
            cp1 = pltpu.async_copy(acat_hbm.at[sidx], asb, sem1)
            cpd.wait()
            cp2 = pltpu.async_copy(acat_hbm.at[didx], adb, sem2)
            cp1.wait()
            cp2.wait()
            for e in range(BB):
                ev = asb[e] + adb[e].at[shift].get(mode="promise_in_bounds")
                ev = jnp.where(ev >= 0, ev, 0.2 * ev)
                wb[e] = jnp.exp(ev)
            pltpu.sync_copy(wb, wt_hbm.at[pl.ds(base, BB)])
            pltpu.sync_copy(wb, den_sh.at[didx], add=True)
            return carry

        lax.fori_loop(0, nblk, blk, 0)
        plsc.subcore_barrier()
        pltpu.sync_copy(den_sh.at[pl.ds(s * slab, slab)],
                        den_hbm.at[c, pl.ds(s * slab, slab)])

    kern = pl.kernel(
        body,
        out_type=[
            jax.ShapeDtypeStruct((EP, 16), jnp.float32),
            jax.ShapeDtypeStruct((2, NP, 16), jnp.float32),
        ],
        mesh=mesh,
        compiler_params=pltpu.CompilerParams(use_tc_tiling_on_sc=False),
        scratch_types=[
            pltpu.VMEM((BB,), jnp.int32),
            pltpu.VMEM((BB,), jnp.int32),
            pltpu.VMEM((BB, 16), jnp.float32),
            pltpu.VMEM((BB, 16), jnp.float32),
            pltpu.VMEM((BB, 16), jnp.float32),
            pltpu.VMEM((BB, 16), jnp.float32),
            pltpu.SemaphoreType.DMA,
            pltpu.SemaphoreType.DMA,
            pltpu.SemaphoreType.DMA,
            pltpu.SemaphoreType.DMA,
            pltpu.VMEM_SHARED((NP, 16), jnp.float32),
        ],
    )
    return kern(acatp, src, dst)


def _sc_agg_heads(hflat, wt, src, dst):
    mesh = plsc.VectorSubcoreMesh(core_axis_name="c", subcore_axis_name="s")
    chunk = EP // 16
    nblk = chunk // BB
    slab = NP // 16

    def body(hflat_hbm, wt_hbm, src_hbm, dst_hbm, num_hbm,
             sidx0, sidx1, didx0, didx1, g00, g01, g10, g11,
             r00, r01, r10, r11, wv0, wv1, zrow,
             si0, si1, sd0, sd1, sw0, sw1, sg00, sg01, sg10, sg11,
             acc0, acc1):
        c = lax.axis_index("c")
        s = lax.axis_index("s")
        sidx = [sidx0, sidx1]
        didx = [didx0, didx1]
        gidx0 = [g00, g01]
        gidx1 = [g10, g11]
        rows0 = [r00, r01]
        rows1 = [r10, r11]
        wv = [wv0, wv1]
        si = [si0, si1]
        sd = [sd0, sd1]
        sw = [sw0, sw1]
        sg0 = [sg00, sg01]
        sg1 = [sg10, sg11]

        def issue_loads(p, b):
            base = s * chunk + b * BB
            pltpu.async_copy(src_hbm.at[pl.ds(base, BB)], sidx[p], si[p])
            pltpu.async_copy(dst_hbm.at[pl.ds(base, BB)], didx[p], sd[p])
            pltpu.async_copy(wt_hbm.at[pl.ds(base, BB)], wv[p], sw[p])

        def wait_sidx(p):
            pltpu.make_async_copy(
                src_hbm.at[pl.ds(0, BB)], sidx[p], si[p]).wait()

        def wait_didx(p):
            pltpu.make_async_copy(
                dst_hbm.at[pl.ds(0, BB)], didx[p], sd[p]).wait()

        def wait_wv(p):
            pltpu.make_async_copy(
                wt_hbm.at[pl.ds(0, BB)], wv[p], sw[p]).wait()

        def fire_gathers(p, off0, off1):
            for i in range(BB // 16):
                sv = sidx[p][pl.ds(i * 16, 16)]
                gidx0[p][pl.ds(i * 16, 16)] = sv + off0
                gidx1[p][pl.ds(i * 16, 16)] = sv + off1
            pltpu.async_copy(hflat_hbm.at[gidx0[p]], rows0[p], sg0[p])
            pltpu.async_copy(hflat_hbm.at[gidx1[p]], rows1[p], sg1[p])

        def wait_g(p, which):
            gi = gidx0[p] if which == 0 else gidx1[p]
            ri = rows0[p] if which == 0 else rows1[p]
            sm = sg0[p] if which == 0 else sg1[p]
            pltpu.make_async_copy(hflat_hbm.at[gi], ri, sm).wait()

        def process(p, h0):
            wait_wv(p)
            wait_g(p, 0)
            for r in range(BB):
                wsp = _bcast(wv[p][r], h0)
                for k in range(HID // 16):
                    rows0[p][r, pl.ds(k * 16, 16)] = (
                        rows0[p][r, pl.ds(k * 16, 16)] * wsp)
            wait_didx(p)
            pltpu.sync_copy(rows0[p], acc0.at[didx[p]], add=True)
            wait_g(p, 1)
            for r in range(BB):
                wsp = _bcast(wv[p][r], h0 + 1)
                for k in range(HID // 16):
                    rows1[p][r, pl.ds(k * 16, 16)] = (
                        rows1[p][r, pl.ds(k * 16, 16)] * wsp)
            pltpu.sync_copy(rows1[p], acc1.at[didx[p]], add=True)

        for r in range(BB):
            for k in range(HID // 16):
                zrow[r, pl.ds(k * 16, 16)] = jnp.zeros((16,), jnp.float32)

        def zero_accs():
            for j in range(slab // BB):
                pltpu.sync_copy(zrow, acc0.at[pl.ds(s * slab + j * BB, BB)])
                pltpu.sync_copy(zrow, acc1.at[pl.ds(s * slab + j * BB, BB)])

        zero_accs()
        plsc.subcore_barrier()
        for hp in range(2):
            h0 = c * 4 + 2 * hp
            off0 = h0 * NN
            off1 = off0 + NN

            issue_loads(0, 0)
            wait_sidx(0)
            fire_gathers(0, off0, off1)
            issue_loads(1, 1)

            def blk(bb, carry):
                e = 2 * bb
                wait_sidx(1)
                fire_gathers(1, off0, off1)
                process(0, h0)
                issue_loads(0, jnp.minimum(e + 2, nblk - 1))
                wait_sidx(0)
                fire_gathers(0, off0, off1)
                process(1, h0)
                issue_loads(1, jnp.minimum(e + 3, nblk - 1))
                return carry

            lax.fori_loop(0, nblk // 2, blk, 0)
            wait_wv(0)
            wait_didx(0)
            wait_g(0, 0)
            wait_g(0, 1)
            wait_sidx(1)
            wait_didx(1)
            wait_wv(1)
            plsc.subcore_barrier()
            pltpu.sync_copy(acc0.at[pl.ds(s * slab, slab)],
                            num_hbm.at[h0, pl.ds(s * slab, slab)])
            pltpu.sync_copy(acc1.at[pl.ds(s * slab, slab)],
                            num_hbm.at[h0 + 1, pl.ds(s * slab, slab)])
            if hp == 0:
                zero_accs()
                plsc.subcore_barrier()

    kern = pl.kernel(
        body,
        out_type=jax.ShapeDtypeStruct((HDS, NP, HID), jnp.float32),
        mesh=mesh,
        compiler_params=pltpu.CompilerParams(use_tc_tiling_on_sc=False),
        scratch_types=(
            [pltpu.VMEM((BB,), jnp.int32) for _ in range(8)]
            + [pltpu.VMEM((BB, HID), jnp.float32) for _ in range(4)]
            + [pltpu.VMEM((BB, 16), jnp.float32) for _ in range(2)]
            + [pltpu.VMEM((BB, HID), jnp.float32)]
            + [pltpu.SemaphoreType.DMA for _ in range(10)]
            + [pltpu.VMEM_SHARED((NP, HID), jnp.float32) for _ in range(2)]
        ),
    )
    return kern(hflat, wt, src, dst)


def _sc_agg_single(h2, wt, src, dst):
    mesh = plsc.VectorSubcoreMesh(core_axis_name="c", subcore_axis_name="s")
    chunk = EP // 32
    nblk = chunk // BB
    slab = NP // 16

    def body(h2_hbm, wt_hbm, src_hbm, dst_hbm, num_hbm,
             sidx, didx, rows, wv, zrow, sem, semi1, semi2, semw, acc_sh):
        c = lax.axis_index("c")
        s = lax.axis_index("s")
        wid = s * 2 + c
        for r in range(BB):
            for k in range(HID // 16):
                zrow[r, pl.ds(k * 16, 16)] = jnp.zeros((16,), jnp.float32)
        for j in range(slab // BB):
            pltpu.sync_copy(zrow, acc_sh.at[pl.ds(s * slab + j * BB, BB)])
        plsc.subcore_barrier()

        def blk(b, carry):
            base = wid * chunk + b * BB
            cps = pltpu.async_copy(src_hbm.at[pl.ds(base, BB)], sidx, semi1)
            cpd = pltpu.async_copy(dst_hbm.at[pl.ds(base, BB)], didx, semi2)
            cpw = pltpu.async_copy(wt_hbm.at[pl.ds(base, BB)], wv, semw)
            cps.wait()
            g0 = pltpu.async_copy(h2_hbm.at[sidx], rows, sem)
            cpw.wait()
            g0.wait()
            for r in range(BB):
                wsp = _bcast(wv[r], 0)
                for k in range(HID // 16):
                    rows[r, pl.ds(k * 16, 16)] = (
                        rows[r, pl.ds(k * 16, 16)] * wsp)
            cpd.wait()
            pltpu.sync_copy(rows, acc_sh.at[didx], add=True)
            return carry

        lax.fori_loop(0, nblk, blk, 0)
        plsc.subcore_barrier()
        pltpu.sync_copy(acc_sh.at[pl.ds(s * slab, slab)],
                        num_hbm.at[c, pl.ds(s * slab, slab)])

    kern = pl.kernel(
        body,
        out_type=jax.ShapeDtypeStruct((2, NP, HID), jnp.float32),
        mesh=mesh,
        compiler_params=pltpu.CompilerParams(use_tc_tiling_on_sc=False),
        scratch_types=[
            pltpu.VMEM((BB,), jnp.int32),
            pltpu.VMEM((BB,), jnp.int32),
            pltpu.VMEM((BB, HID), jnp.float32),
            pltpu.VMEM((BB, 16), jnp.float32),
            pltpu.VMEM((BB, HID), jnp.float32),
            pltpu.SemaphoreType.DMA,
            pltpu.SemaphoreType.DMA,
            pltpu.SemaphoreType.DMA,
            pltpu.SemaphoreType.DMA,
            pltpu.VMEM_SHARED((NP, HID), jnp.float32),
        ],
    )
    return kern(h2, wt, src, dst)


def kernel(x, edge_index, W1, a_src1, a_dst1, b1, g1, be1,
           W2, a_src2, a_dst2, b2, g2, be2, mW1, mb1, mW2, mb2):
    src0 = edge_index[0].astype(jnp.int32)
    dst0 = edge_index[1].astype(jnp.int32)
    loop = jnp.arange(NN, dtype=jnp.int32)
    npad = EP - (EE + NN)
    src = jnp.concatenate([src0, loop, jnp.zeros((npad,), jnp.int32)])
    dst = jnp.concatenate([dst0, loop, jnp.full((npad,), NN, jnp.int32)])

    h1t, acat1 = _tc1(x, W1, a_src1, a_dst1)
    acat1p = jnp.pad(acat1, ((0, NP - NN), (0, 0)))
    w1t, den1 = _sc_weights(acat1p, src, dst, HDS)
    num1 = _sc_agg_heads(h1t.reshape(HDS * NN, HID), w1t, src, dst)

    h2, acat2 = _tc2(num1, den1, b1, g1, be1, W2, a_src2, a_dst2)
    acat2p = jnp.pad(acat2, ((0, NP - NN), (0, 0)))
    w2t, den2 = _sc_weights(acat2p, src, dst, 1)
    num2 = _sc_agg_single(h2, w2t, src, dst)

    y = _tc3(num2, den2, b2, g2, be2, x, mW1, mb1,
             mW2.reshape(HID), mb2)
    return y[:NN]


# speedup vs baseline: 31.4199x; 1.1061x over previous
import functools
import math

import jax
import jax.numpy as jnp
from jax import lax
from jax.experimental import pallas as pl
from jax.experimental.pallas import tpu as pltpu
from jax.experimental.pallas import tpu_sc as plsc

NN = 10000
NP = 10240
EE = 320000
EP = 331776
BB = 128
DIN = 128
HID = 64
HDS = 8
RSIG = float(1.0 / math.sqrt(1.0 + 1e-5))


def _elu(t):
    return jnp.where(t > 0, t, jnp.exp(jnp.minimum(t, 0.0)) - 1.0)


def _tc1_body(x_ref, w1_ref, as_ref, ad_ref, h_out, acat_out):
    hb = jnp.dot(x_ref[...], w1_ref[...], preferred_element_type=jnp.float32)
    iota = lax.broadcasted_iota(jnp.int32, (1, 16), 1)
    acat = jnp.zeros((x_ref.shape[0], 16), jnp.float32)
    for h in range(HDS):
        hh = hb[:, h * HID:(h + 1) * HID]
        h_out[h] = hh
        asv = jnp.sum(hh * as_ref[h][None, :], axis=1)
        adv = jnp.sum(hh * ad_ref[h][None, :], axis=1)
        acat = acat + jnp.where(iota == h, asv[:, None], 0.0)
        acat = acat + jnp.where(iota == HDS + h, adv[:, None], 0.0)
    acat_out[...] = acat


def _tc1(x, W1, a_src1, a_dst1):
    nb = NP // 512
    return pl.pallas_call(
        _tc1_body,
        grid=(nb,),
        in_specs=[
            pl.BlockSpec((512, DIN), lambda i: (i, 0)),
            pl.BlockSpec((DIN, HDS * HID), lambda i: (0, 0)),
            pl.BlockSpec((HDS, HID), lambda i: (0, 0)),
            pl.BlockSpec((HDS, HID), lambda i: (0, 0)),
        ],
        out_specs=[
            pl.BlockSpec((HDS, 512, HID), lambda i: (0, i, 0)),
            pl.BlockSpec((512, 16), lambda i: (i, 0)),
        ],
        out_shape=[
            jax.ShapeDtypeStruct((HDS, NN, HID), jnp.float32),
            jax.ShapeDtypeStruct((NN, 16), jnp.float32),
        ],
    )(x, W1, a_src1, a_dst1)


def _tc2_body(num_ref, den_ref, b1_ref, g1_ref, be1_ref, w2_ref,
              as2_ref, ad2_ref, h2_out, acat_out):
    den = den_ref[0] + den_ref[1]
    acc = jnp.zeros((num_ref.shape[1], HID), jnp.float32)
    for h in range(HDS):
        t = num_ref[h] / (den[:, h][:, None] + 1e-16)
        t = t + b1_ref[h * HID:(h + 1) * HID][None, :]
        t = t * (g1_ref[h * HID:(h + 1) * HID][None, :] * RSIG)
        t = t + be1_ref[h * HID:(h + 1) * HID][None, :]
        t = _elu(t)
        acc = acc + jnp.dot(t, w2_ref[h * HID:(h + 1) * HID, :],
                            preferred_element_type=jnp.float32)
    h2_out[...] = acc
    asv = jnp.sum(acc * as2_ref[...], axis=1)
    adv = jnp.sum(acc * ad2_ref[...], axis=1)
    iota = lax.broadcasted_iota(jnp.int32, (1, 16), 1)
    acat_out[...] = (jnp.where(iota == 0, asv[:, None], 0.0)
                     + jnp.where(iota == 1, adv[:, None], 0.0))


def _tc2(num1, den1, b1, g1, be1, W2, a_src2, a_dst2):
    nb = NP // 512
    return pl.pallas_call(
        _tc2_body,
        grid=(nb,),
        in_specs=[
            pl.BlockSpec((HDS, 512, HID), lambda i: (0, i, 0)),
            pl.BlockSpec((2, 512, 16), lambda i: (0, i, 0)),
            pl.BlockSpec((HDS * HID,), lambda i: (0,)),
            pl.BlockSpec((HDS * HID,), lambda i: (0,)),
            pl.BlockSpec((HDS * HID,), lambda i: (0,)),
            pl.BlockSpec((HDS * HID, HID), lambda i: (0, 0)),
            pl.BlockSpec((1, HID), lambda i: (0, 0)),
            pl.BlockSpec((1, HID), lambda i: (0, 0)),
        ],
        out_specs=[
            pl.BlockSpec((512, HID), lambda i: (i, 0)),
            pl.BlockSpec((512, 16), lambda i: (i, 0)),
        ],
        out_shape=[
            jax.ShapeDtypeStruct((NN, HID), jnp.float32),
            jax.ShapeDtypeStruct((NN, 16), jnp.float32),
        ],
    )(num1, den1, b1, g1, be1, W2, a_src2, a_dst2)


def _tc3_body(num_ref, den_ref, b2_ref, g2_ref, be2_ref, x_ref,
              mw1_ref, mb1_ref, mw2_ref, mb2_ref, y_out):
    num = num_ref[0] + num_ref[1]
    den = den_ref[0, :, 0] + den_ref[1, :, 0]
    t = num / (den[:, None] + 1e-16) + b2_ref[...][None, :]
    t = t * (g2_ref[...][None, :] * RSIG) + be2_ref[...][None, :]
    t = _elu(t)
    m = (jnp.dot(t, mw1_ref[0:HID, :], preferred_element_type=jnp.float32)
         + jnp.dot(x_ref[...], mw1_ref[HID:HID + DIN, :],
                   preferred_element_type=jnp.float32)
         + mb1_ref[...][None, :])
    m = jnp.maximum(m, 0.0)
    y = jnp.sum(m * mw2_ref[...][None, :], axis=1) + mb2_ref[...]
    y_out[...] = y


def _tc3(num2, den2, b2, g2, be2, x, mW1, mb1, mw2v, mb2):
    nb = NP // 512
    return pl.pallas_call(
        _tc3_body,
        grid=(nb,),
        in_specs=[
            pl.BlockSpec((2, 512, HID), lambda i: (0, i, 0)),
            pl.BlockSpec((2, 512, 16), lambda i: (0, i, 0)),
            pl.BlockSpec((HID,), lambda i: (0,)),
            pl.BlockSpec((HID,), lambda i: (0,)),
            pl.BlockSpec((HID,), lambda i: (0,)),
            pl.BlockSpec((512, DIN), lambda i: (i, 0)),
            pl.BlockSpec((HID + DIN, HID), lambda i: (0, 0)),
            pl.BlockSpec((HID,), lambda i: (0,)),
            pl.BlockSpec((HID,), lambda i: (0,)),
            pl.BlockSpec((1,), lambda i: (0,)),
        ],
        out_specs=pl.BlockSpec((512,), lambda i: (i,)),
        out_shape=jax.ShapeDtypeStruct((NP,), jnp.float32),
    )(num2, den2, b2, g2, be2, x, mW1, mb1, mw2v, mb2)


def _bcast(v, lane):
    idx = jnp.full((16,), lane, jnp.int32)
    return v.at[idx].get(mode="promise_in_bounds")


def _sc_weights(acatp, src, dst, heads):
    mesh = plsc.VectorSubcoreMesh(core_axis_name="c", subcore_axis_name="s")
    chunk = EP // 32
    nblk = chunk // BB
    slab = NP // 16

    def body(acat_hbm, src_hbm, dst_hbm, wt_hbm, den_hbm,
             sidx, didx, asb, adb, wb, zrw, sem1, sem2, semi1, semi2, den_sh):
        c = lax.axis_index("c")
        s = lax.axis_index("s")
        wid = s * 2 + c
        for r in range(BB):
            zrw[r] = jnp.zeros((16,), jnp.float32)
        for j in range(slab // BB):
            pltpu.sync_copy(zrw, den_sh.at[pl.ds(s * slab + j * BB, BB)])
        plsc.subcore_barrier()
        shift = (jnp.arange(16, dtype=jnp.int32) % heads) + heads

        def blk(b, carry):
            base = wid * chunk + b * BB
            cps = pltpu.async_copy(src_hbm.at[pl.ds(base, BB)], sidx, semi1)
            cpd = pltpu.async_copy(dst_hbm.at[pl.ds(base, BB)], didx, semi2)
            cps.wait()
            cp1 = pltpu.async_copy(acat_hbm.at[sidx], asb, sem1)
            cpd.wait()
            cp2 = pltpu.async_copy(acat_hbm.at[didx], adb, sem2)
            cp1.wait()
            cp2.wait()
            for e in range(BB):
                ev = asb[e] + adb[e].at[shift].get(mode="promise_in_bounds")
                ev = jnp.where(ev >= 0, ev, 0.2 * ev)
                wb[e] = jnp.exp(ev)
            pltpu.sync_copy(wb, wt_hbm.at[pl.ds(base, BB)])
            pltpu.sync_copy(wb, den_sh.at[didx], add=True)
            return carry

        lax.fori_loop(0, nblk, blk, 0)
        plsc.subcore_barrier()
        pltpu.sync_copy(den_sh.at[pl.ds(s * slab, slab)],
                        den_hbm.at[c, pl.ds(s * slab, slab)])

    kern = pl.kernel(
        body,
        out_type=[
            jax.ShapeDtypeStruct((EP, 16), jnp.float32),
            jax.ShapeDtypeStruct((2, NP, 16), jnp.float32),
        ],
        mesh=mesh,
        compiler_params=pltpu.CompilerParams(use_tc_tiling_on_sc=False),
        scratch_types=[
            pltpu.VMEM((BB,), jnp.int32),
            pltpu.VMEM((BB,), jnp.int32),
            pltpu.VMEM((BB, 16), jnp.float32),
            pltpu.VMEM((BB, 16), jnp.float32),
            pltpu.VMEM((BB, 16), jnp.float32),
            pltpu.VMEM((BB, 16), jnp.float32),
            pltpu.SemaphoreType.DMA,
            pltpu.SemaphoreType.DMA,
            pltpu.SemaphoreType.DMA,
            pltpu.SemaphoreType.DMA,
            pltpu.VMEM_SHARED((NP, 16), jnp.float32),
        ],
    )
    return kern(acatp, src, dst)


def _sc_agg_heads(hflat, wt, src, dst):
    mesh = plsc.VectorSubcoreMesh(core_axis_name="c", subcore_axis_name="s")
    chunk = EP // 16
    nblk = chunk // BB
    slab = NP // 16

    def body(hflat_hbm, wt_hbm, src_hbm, dst_hbm, num_hbm,
             sidx0, sidx1, didx0, didx1, g00, g01, g10, g11,
             r00, r01, r10, r11, wv0, wv1, zrow,
             si0, si1, sd0, sd1, sw0, sw1, sg00, sg01, sg10, sg11,
             ssa0, ssa1, ssb0, ssb1, acc0, acc1):
        c = lax.axis_index("c")
        s = lax.axis_index("s")
        sidx = [sidx0, sidx1]
        didx = [didx0, didx1]
        gidx0 = [g00, g01]
        gidx1 = [g10, g11]
        rows0 = [r00, r01]
        rows1 = [r10, r11]
        wv = [wv0, wv1]
        si = [si0, si1]
        sd = [sd0, sd1]
        sw = [sw0, sw1]
        sg0 = [sg00, sg01]
        sg1 = [sg10, sg11]
        ss0 = [ssa0, ssa1]
        ss1 = [ssb0, ssb1]

        def issue_sw(p, b):
            base = s * chunk + b * BB
            pltpu.async_copy(src_hbm.at[pl.ds(base, BB)], sidx[p], si[p])
            pltpu.async_copy(wt_hbm.at[pl.ds(base, BB)], wv[p], sw[p])

        def issue_didx(p, b):
            base = s * chunk + b * BB
            pltpu.async_copy(dst_hbm.at[pl.ds(base, BB)], didx[p], sd[p])

        def wait_sidx(p):
            pltpu.make_async_copy(
                src_hbm.at[pl.ds(0, BB)], sidx[p], si[p]).wait()

        def wait_didx(p):
            pltpu.make_async_copy(
                dst_hbm.at[pl.ds(0, BB)], didx[p], sd[p]).wait()

        def wait_wv(p):
            pltpu.make_async_copy(
                wt_hbm.at[pl.ds(0, BB)], wv[p], sw[p]).wait()

        def fire_gathers(p, off0, off1):
            for i in range(BB // 16):
                sv = sidx[p][pl.ds(i * 16, 16)]
                gidx0[p][pl.ds(i * 16, 16)] = sv + off0
                gidx1[p][pl.ds(i * 16, 16)] = sv + off1
            pltpu.async_copy(hflat_hbm.at[gidx0[p]], rows0[p], sg0[p])
            pltpu.async_copy(hflat_hbm.at[gidx1[p]], rows1[p], sg1[p])

        def wait_g(p, which):
            gi = gidx0[p] if which == 0 else gidx1[p]
            ri = rows0[p] if which == 0 else rows1[p]
            sm = sg0[p] if which == 0 else sg1[p]
            pltpu.make_async_copy(hflat_hbm.at[gi], ri, sm).wait()

        def process(p, h0):
            wait_wv(p)
            wait_g(p, 0)
            for r in range(BB):
                wsp = _bcast(wv[p][r], h0)
                for k in range(HID // 16):
                    rows0[p][r, pl.ds(k * 16, 16)] = (
                        rows0[p][r, pl.ds(k * 16, 16)] * wsp)
            wait_didx(p)
            pltpu.async_copy(rows0[p], acc0.at[didx[p]], ss0[p], add=True)
            wait_g(p, 1)
            for r in range(BB):
                wsp = _bcast(wv[p][r], h0 + 1)
                for k in range(HID // 16):
                    rows1[p][r, pl.ds(k * 16, 16)] = (
                        rows1[p][r, pl.ds(k * 16, 16)] * wsp)
            pltpu.async_copy(rows1[p], acc1.at[didx[p]], ss1[p], add=True)

        def wait_sc(p):
            pltpu.make_async_copy(rows0[p], acc0.at[didx[p]], ss0[p]).wait()
            pltpu.make_async_copy(rows1[p], acc1.at[didx[p]], ss1[p]).wait()

        for r in range(BB):
            for k in range(HID // 16):
                zrow[r, pl.ds(k * 16, 16)] = jnp.zeros((16,), jnp.float32)

        def zero_accs():
            for j in range(slab // BB):
                pltpu.sync_copy(zrow, acc0.at[pl.ds(s * slab + j * BB, BB)])
                pltpu.sync_copy(zrow, acc1.at[pl.ds(s * slab + j * BB, BB)])

        zero_accs()
        plsc.subcore_barrier()
        for hp in range(2):
            h0 = c * 4 + 2 * hp
            off0 = h0 * NN
            off1 = off0 + NN

            issue_sw(0, 0)
            issue_didx(0, 0)
            wait_sidx(0)
            fire_gathers(0, off0, off1)
            issue_sw(1, 1)

            def blk(bb, carry):
                e2 = jnp.minimum(2 * bb + 2, nblk - 1)
                o2 = jnp.minimum(2 * bb + 3, nblk - 1)

                @pl.when(bb > 0)
                def _():
                    wait_sc(1)

                issue_didx(1, 2 * bb + 1)
                wait_sidx(1)
                fire_gathers(1, off0, off1)
                process(0, h0)
                issue_sw(0, e2)
                wait_sc(0)
                issue_didx(0, e2)
                wait_sidx(0)
                fire_gathers(0, off0, off1)
                process(1, h0)
                issue_sw(1, o2)
                return carry

            lax.fori_loop(0, nblk // 2, blk, 0)
            wait_wv(0)
            wait_didx(0)
            wait_g(0, 0)
            wait_g(0, 1)
            wait_sidx(1)
            wait_wv(1)
            wait_sc(1)
            plsc.subcore_barrier()
            pltpu.sync_copy(acc0.at[pl.ds(s * slab, slab)],
                            num_hbm.at[h0, pl.ds(s * slab, slab)])
            pltpu.sync_copy(acc1.at[pl.ds(s * slab, slab)],
                            num_hbm.at[h0 + 1, pl.ds(s * slab, slab)])
            if hp == 0:
                zero_accs()
                plsc.subcore_barrier()

    kern = pl.kernel(
        body,
        out_type=jax.ShapeDtypeStruct((HDS, NP, HID), jnp.float32),
        mesh=mesh,
        compiler_params=pltpu.CompilerParams(use_tc_tiling_on_sc=False),
        scratch_types=(
            [pltpu.VMEM((BB,), jnp.int32) for _ in range(8)]
            + [pltpu.VMEM((BB, HID), jnp.float32) for _ in range(4)]
            + [pltpu.VMEM((BB, 16), jnp.float32) for _ in range(2)]
            + [pltpu.VMEM((BB, HID), jnp.float32)]
            + [pltpu.SemaphoreType.DMA for _ in range(14)]
            + [pltpu.VMEM_SHARED((NP, HID), jnp.float32) for _ in range(2)]
        ),
    )
    return kern(hflat, wt, src, dst)


def _sc_agg_single(h2, wt, src, dst):
    mesh = plsc.VectorSubcoreMesh(core_axis_name="c", subcore_axis_name="s")
    chunk = EP // 32
    nblk = chunk // BB
    slab = NP // 16

    def body(h2_hbm, wt_hbm, src_hbm, dst_hbm, num_hbm,
             sidx, didx, rows, wv, zrow, sem, semi1, semi2, semw, acc_sh):
        c = lax.axis_index("c")
        s = lax.axis_index("s")
        wid = s * 2 + c
        for r in range(BB):
            for k in range(HID // 16):
                zrow[r, pl.ds(k * 16, 16)] = jnp.zeros((16,), jnp.float32)
        for j in range(slab // BB):
            pltpu.sync_copy(zrow, acc_sh.at[pl.ds(s * slab + j * BB, BB)])
        plsc.subcore_barrier()

        def blk(b, carry):
            base = wid * chunk + b * BB
            cps = pltpu.async_copy(src_hbm.at[pl.ds(base, BB)], sidx, semi1)
            cpd = pltpu.async_copy(dst_hbm.at[pl.ds(base, BB)], didx, semi2)
            cpw = pltpu.async_copy(wt_hbm.at[pl.ds(base, BB)], wv, semw)
            cps.wait()
            g0 = pltpu.async_copy(h2_hbm.at[sidx], rows, sem)
            cpw.wait()
            g0.wait()
            for r in range(BB):
                wsp = _bcast(wv[r], 0)
                for k in range(HID // 16):
                    rows[r, pl.ds(k * 16, 16)] = (
                        rows[r, pl.ds(k * 16, 16)] * wsp)
            cpd.wait()
            pltpu.sync_copy(rows, acc_sh.at[didx], add=True)
            return carry

        lax.fori_loop(0, nblk, blk, 0)
        plsc.subcore_barrier()
        pltpu.sync_copy(acc_sh.at[pl.ds(s * slab, slab)],
                        num_hbm.at[c, pl.ds(s * slab, slab)])

    kern = pl.kernel(
        body,
        out_type=jax.ShapeDtypeStruct((2, NP, HID), jnp.float32),
        mesh=mesh,
        compiler_params=pltpu.CompilerParams(use_tc_tiling_on_sc=False),
        scratch_types=[
            pltpu.VMEM((BB,), jnp.int32),
            pltpu.VMEM((BB,), jnp.int32),
            pltpu.VMEM((BB, HID), jnp.float32),
            pltpu.VMEM((BB, 16), jnp.float32),
            pltpu.VMEM((BB, HID), jnp.float32),
            pltpu.SemaphoreType.DMA,
            pltpu.SemaphoreType.DMA,
            pltpu.SemaphoreType.DMA,
            pltpu.SemaphoreType.DMA,
            pltpu.VMEM_SHARED((NP, HID), jnp.float32),
        ],
    )
    return kern(h2, wt, src, dst)


def kernel(x, edge_index, W1, a_src1, a_dst1, b1, g1, be1,
           W2, a_src2, a_dst2, b2, g2, be2, mW1, mb1, mW2, mb2):
    src0 = edge_index[0].astype(jnp.int32)
    dst0 = edge_index[1].astype(jnp.int32)
    loop = jnp.arange(NN, dtype=jnp.int32)
    npad = EP - (EE + NN)
    src = jnp.concatenate([src0, loop, jnp.zeros((npad,), jnp.int32)])
    dst = jnp.concatenate([dst0, loop, jnp.full((npad,), NN, jnp.int32)])

    h1t, acat1 = _tc1(x, W1, a_src1, a_dst1)
    acat1p = jnp.pad(acat1, ((0, NP - NN), (0, 0)))
    w1t, den1 = _sc_weights(acat1p, src, dst, HDS)
    num1 = _sc_agg_heads(h1t.reshape(HDS * NN, HID), w1t, src, dst)

    h2, acat2 = _tc2(num1, den1, b1, g1, be1, W2, a_src2, a_dst2)
    acat2p = jnp.pad(acat2, ((0, NP - NN), (0, 0)))
    w2t, den2 = _sc_weights(acat2p, src, dst, 1)
    num2 = _sc_agg_single(h2, w2t, src, dst)

    y = _tc3(num2, den2, b2, g2, be2, x, mW1, mb1,
             mW2.reshape(HID), mb2)
    return y[:NN]
